# baseline (speedup 1.0000x reference)
import jax
import jax.numpy as jnp
from jax.experimental import pallas as pl

DEPTH_ = 4
AF_ = 128
BF_ = 16
H_ = 512
NA_ = 128
NB_ = 256
K_ = 6


def _wln_body(af_ref, bf_ref, ag_ref, bg_ref, rev_ref, mn_ref, ma_ref,
              w1a_ref, w1b_ref, wa_ref, wb_ref, watb_ref, wa2_ref, wb2_ref,
              bnei_ref, batom_ref, bbond_ref,
              out_a_ref, out_b_ref):
    f32 = jnp.float32
    a = jnp.maximum(jnp.dot(af_ref[0], w1a_ref[...]), 0.0)
    b = jnp.maximum(jnp.dot(bf_ref[0], w1b_ref[...]), 0.0)

    ag = ag_ref[0]
    bg = bg_ref[0]
    rev = rev_ref[0]
    mn = mn_ref[0]
    ma = ma_ref[0]

    iota_a = jax.lax.broadcasted_iota(jnp.int32, (NA_, NA_), 1)
    iota_b = jax.lax.broadcasted_iota(jnp.int32, (NA_, NB_), 1)
    iota_r = jax.lax.broadcasted_iota(jnp.int32, (NB_, NA_), 1)

    PaS = jnp.concatenate(
        [(ag[:, k:k + 1] == iota_a).astype(f32) for k in range(K_)], axis=0)
    PbS = jnp.concatenate(
        [(bg[:, k:k + 1] == iota_b).astype(f32) for k in range(K_)], axis=0)
    P2 = jnp.concatenate([PaS, PbS], axis=1)
    Pr0 = (rev[:, 0:1] == iota_r).astype(f32)
    Pr1 = (rev[:, 1:2] == iota_r).astype(f32)
    Prs = Pr0 + Pr1

    bnei = bnei_ref[...]
    batom = batom_ref[...]
    bbond = bbond_ref[...]

    for _ in range(DEPTH_ - 1):
        acat = jnp.dot(a, wa_ref[...])
        bcat = jnp.dot(b, wb_ref[...])
        T = jnp.concatenate([acat[:, :H_], bcat[:, :H_]], axis=0)
        G = mn * jnp.maximum(jnp.dot(P2, T) + bnei, 0.0)
        ann = G[:NA_]
        for k in range(1, K_):
            ann = ann + G[k * NA_:(k + 1) * NA_]
        a_new = jnp.maximum(acat[:, H_:2 * H_] + jnp.dot(ann, watb_ref[...]) + batom, 0.0)
        b_new = jnp.maximum(bcat[:, H_:] + jnp.dot(Prs, acat[:, 2 * H_:]) + bbond, 0.0)
        a, b = a_new, b_new

    acat = jnp.dot(a, wa2_ref[...])
    bcat = jnp.dot(b, wb2_ref[...])
    G = mn * (jnp.dot(PaS, acat[:, :H_]) * jnp.dot(PbS, bcat[:, :H_]))
    ann = G[:NA_]
    for k in range(1, K_):
        ann = ann + G[k * NA_:(k + 1) * NA_]
    out_a_ref[0] = ma * (acat[:, H_:2 * H_] * ann)
    aWb = acat[:, 2 * H_:]
    out_b_ref[0] = jnp.dot(Pr0, aWb) * jnp.dot(Pr1, aWb) * bcat[:, H_:]


def kernel(atom_feats, bond_feats, atom_graph, bond_graph, rev_atom_graph,
           mask_neis, mask_atoms, W1a, W1b, Wnei, bnei, Watom, batom,
           Wbond, bbond, W2a_atom, W2a_bond, W2b_atom, W2a, W2b):
    B = atom_feats.shape[0]
    f32 = jnp.float32
    Wa_cat = jnp.concatenate([Wnei[:H_], Watom[:H_], Wbond[H_:]], axis=1)
    Wb_cat = jnp.concatenate([Wnei[H_:], Wbond[:H_]], axis=1)
    Watom_b = Watom[H_:]
    Wa2_cat = jnp.concatenate([W2a_atom, W2a, W2b_atom], axis=1)
    Wb2_cat = jnp.concatenate([W2a_bond, W2b], axis=1)
    mn = mask_neis.astype(f32).reshape(B, NA_, K_).transpose(0, 2, 1).reshape(B, NA_ * K_, 1)
    ma = mask_atoms.astype(f32)
    bnei2 = bnei.reshape(1, H_)
    batom2 = batom.reshape(1, H_)
    bbond2 = bbond.reshape(1, H_)

    def im_g(i):
        return (i, 0, 0)

    def im_w(i):
        return (0, 0)

    out = pl.pallas_call(
        _wln_body,
        grid=(B,),
        in_specs=[
            pl.BlockSpec((1, NA_, AF_), im_g),
            pl.BlockSpec((1, NB_, BF_), im_g),
            pl.BlockSpec((1, NA_, K_), im_g),
            pl.BlockSpec((1, NA_, K_), im_g),
            pl.BlockSpec((1, NB_, 2), im_g),
            pl.BlockSpec((1, NA_ * K_, 1), im_g),
            pl.BlockSpec((1, NA_, 1), im_g),
            pl.BlockSpec((AF_, H_), im_w),
            pl.BlockSpec((BF_, H_), im_w),
            pl.BlockSpec((H_, 3 * H_), im_w),
            pl.BlockSpec((H_, 2 * H_), im_w),
            pl.BlockSpec((H_, H_), im_w),
            pl.BlockSpec((H_, 3 * H_), im_w),
            pl.BlockSpec((H_, 2 * H_), im_w),
            pl.BlockSpec((1, H_), im_w),
            pl.BlockSpec((1, H_), im_w),
            pl.BlockSpec((1, H_), im_w),
        ],
        out_specs=(
            pl.BlockSpec((1, NA_, H_), im_g),
            pl.BlockSpec((1, NB_, H_), im_g),
        ),
        out_shape=(
            jax.ShapeDtypeStruct((B, NA_, H_), f32),
            jax.ShapeDtypeStruct((B, NB_, H_), f32),
        ),
    )(atom_feats, bond_feats, atom_graph, bond_graph, rev_atom_graph,
      mn, ma, W1a, W1b, Wa_cat, Wb_cat, Watom_b, Wa2_cat, Wb2_cat,
      bnei2, batom2, bbond2)
    return out


# speedup vs baseline: 1.0592x; 1.0592x over previous
import jax
import jax.numpy as jnp
from jax.experimental import pallas as pl

DEPTH_ = 4
AF_ = 128
BF_ = 16
H_ = 512
NA_ = 128
NB_ = 256
K_ = 6


def _wln_body(af_ref, bf_ref, ag_ref, bg_ref, rev_ref, mn_ref, ma_ref,
              w1a_ref, w1b_ref, wa_ref, wb_ref, watb_ref, wa2_ref, wb2_ref,
              bnei_ref, batom_ref, bbond_ref,
              out_a_ref, out_b_ref):
    f32 = jnp.float32
    a = jnp.maximum(jnp.dot(af_ref[0], w1a_ref[...]), 0.0)
    b = jnp.maximum(jnp.dot(bf_ref[0], w1b_ref[...]), 0.0)

    ag = ag_ref[0]
    bg = bg_ref[0]
    rev = rev_ref[0]
    mn = mn_ref[0]
    ma = ma_ref[0]

    iota_a = jax.lax.broadcasted_iota(jnp.int32, (NA_, NA_), 1)
    iota_b = jax.lax.broadcasted_iota(jnp.int32, (NA_, NB_), 1)
    iota_r = jax.lax.broadcasted_iota(jnp.int32, (NB_, NA_), 1)

    Pa = [(ag[:, k:k + 1] == iota_a).astype(f32) for k in range(K_)]
    Pb = [(bg[:, k:k + 1] == iota_b).astype(f32) for k in range(K_)]
    Pr0 = (rev[:, 0:1] == iota_r).astype(f32)
    Pr1 = (rev[:, 1:2] == iota_r).astype(f32)
    Prs = Pr0 + Pr1

    bnei = bnei_ref[...]
    batom = batom_ref[...]
    bbond = bbond_ref[...]

    for _ in range(DEPTH_ - 1):
        acat = jnp.dot(a, wa_ref[...])
        bcat = jnp.dot(b, wb_ref[...])
        aW = acat[:, :H_]
        bW = bcat[:, :H_]
        ann = jnp.zeros((NA_, H_), f32)
        for k in range(K_):
            gk = jnp.dot(Pa[k], aW) + jnp.dot(Pb[k], bW)
            ann = ann + mn[:, k:k + 1] * jnp.maximum(gk + bnei, 0.0)
        a_new = jnp.maximum(acat[:, H_:2 * H_] + jnp.dot(ann, watb_ref[...]) + batom, 0.0)
        b_new = jnp.maximum(bcat[:, H_:] + jnp.dot(Prs, acat[:, 2 * H_:]) + bbond, 0.0)
        a, b = a_new, b_new

    acat = jnp.dot(a, wa2_ref[...])
    bcat = jnp.dot(b, wb2_ref[...])
    aW = acat[:, :H_]
    bW = bcat[:, :H_]
    ann = jnp.zeros((NA_, H_), f32)
    for k in range(K_):
        ann = ann + mn[:, k:k + 1] * (jnp.dot(Pa[k], aW) * jnp.dot(Pb[k], bW))
    out_a_ref[0] = ma * (acat[:, H_:2 * H_] * ann)
    aWb = acat[:, 2 * H_:]
    out_b_ref[0] = jnp.dot(Pr0, aWb) * jnp.dot(Pr1, aWb) * bcat[:, H_:]


def kernel(atom_feats, bond_feats, atom_graph, bond_graph, rev_atom_graph,
           mask_neis, mask_atoms, W1a, W1b, Wnei, bnei, Watom, batom,
           Wbond, bbond, W2a_atom, W2a_bond, W2b_atom, W2a, W2b):
    B = atom_feats.shape[0]
    f32 = jnp.float32
    Wa_cat = jnp.concatenate([Wnei[:H_], Watom[:H_], Wbond[H_:]], axis=1)
    Wb_cat = jnp.concatenate([Wnei[H_:], Wbond[:H_]], axis=1)
    Watom_b = Watom[H_:]
    Wa2_cat = jnp.concatenate([W2a_atom, W2a, W2b_atom], axis=1)
    Wb2_cat = jnp.concatenate([W2a_bond, W2b], axis=1)
    mn = mask_neis.astype(f32).reshape(B, NA_, K_)
    ma = mask_atoms.astype(f32)
    bnei2 = bnei.reshape(1, H_)
    batom2 = batom.reshape(1, H_)
    bbond2 = bbond.reshape(1, H_)

    def im_g(i):
        return (i, 0, 0)

    def im_w(i):
        return (0, 0)

    out = pl.pallas_call(
        _wln_body,
        grid=(B,),
        in_specs=[
            pl.BlockSpec((1, NA_, AF_), im_g),
            pl.BlockSpec((1, NB_, BF_), im_g),
            pl.BlockSpec((1, NA_, K_), im_g),
            pl.BlockSpec((1, NA_, K_), im_g),
            pl.BlockSpec((1, NB_, 2), im_g),
            pl.BlockSpec((1, NA_, K_), im_g),
            pl.BlockSpec((1, NA_, 1), im_g),
            pl.BlockSpec((AF_, H_), im_w),
            pl.BlockSpec((BF_, H_), im_w),
            pl.BlockSpec((H_, 3 * H_), im_w),
            pl.BlockSpec((H_, 2 * H_), im_w),
            pl.BlockSpec((H_, H_), im_w),
            pl.BlockSpec((H_, 3 * H_), im_w),
            pl.BlockSpec((H_, 2 * H_), im_w),
            pl.BlockSpec((1, H_), im_w),
            pl.BlockSpec((1, H_), im_w),
            pl.BlockSpec((1, H_), im_w),
        ],
        out_specs=(
            pl.BlockSpec((1, NA_, H_), im_g),
            pl.BlockSpec((1, NB_, H_), im_g),
        ),
        out_shape=(
            jax.ShapeDtypeStruct((B, NA_, H_), f32),
            jax.ShapeDtypeStruct((B, NB_, H_), f32),
        ),
    )(atom_feats, bond_feats, atom_graph, bond_graph, rev_atom_graph,
      mn, ma, W1a, W1b, Wa_cat, Wb_cat, Watom_b, Wa2_cat, Wb2_cat,
      bnei2, batom2, bbond2)
    return out
